# Initial kernel scaffold; baseline (speedup 1.0000x reference)
import jax
import jax.numpy as jnp
from jax.experimental import pallas as pl


def kernel(x, edge_index, batch, W1a, b1a, W1b, b1b, W2a, b2a, W2b, b2b, Wf, bf):
    raise NotImplementedError("write your pallas kernel here")


# speedup vs baseline: 6.4510x; 6.4510x over previous
import functools

import jax
import jax.numpy as jnp
from jax import lax
from jax.experimental import pallas as pl
from jax.experimental.pallas import tpu as pltpu
from jax.experimental.pallas import tpu_sc as plsc

N = 10000
E = 320000
D = 128
H = 128
OUT = 10
G = 64

NTILES = 32
EPT = E // NTILES
CHUNK = 80
NCH = EPT // CHUNK
RPT = 632
RPT_LAST = N - 15 * RPT

_mesh = plsc.VectorSubcoreMesh(core_axis_name="c", subcore_axis_name="s")


@functools.partial(
    pl.kernel,
    out_type=jax.ShapeDtypeStruct((2, N, D), jnp.float32),
    mesh=_mesh,
    scratch_types=[
        pltpu.VMEM((NCH, CHUNK), jnp.int32),
        pltpu.VMEM((NCH, CHUNK), jnp.int32),
        pltpu.VMEM((CHUNK, D), jnp.float32),
        pltpu.VMEM_SHARED((N, D), jnp.float32),
        pltpu.SemaphoreType.DMA,
    ],
)
def _scatter_agg(x_hbm, src_hbm, dst_hbm, zeros_hbm, out_hbm,
                 src_v, dst_v, rows_v, acc, sem):
    c = lax.axis_index("c")
    s = lax.axis_index("s")
    wid = c * 16 + s

    off = pl.multiple_of(s * RPT, 8)

    @pl.when((c == 0) & (s < 15))
    def _():
        pltpu.sync_copy(x_hbm.at[pl.ds(off, RPT)], acc.at[pl.ds(off, RPT)])

    @pl.when((c == 0) & (s == 15))
    def _():
        pltpu.sync_copy(x_hbm.at[pl.ds(15 * RPT, RPT_LAST)],
                        acc.at[pl.ds(15 * RPT, RPT_LAST)])

    @pl.when((c != 0) & (s < 15))
    def _():
        pltpu.sync_copy(zeros_hbm.at[pl.ds(0, RPT)], acc.at[pl.ds(off, RPT)])

    @pl.when((c != 0) & (s == 15))
    def _():
        pltpu.sync_copy(zeros_hbm.at[pl.ds(0, RPT_LAST)],
                        acc.at[pl.ds(15 * RPT, RPT_LAST)])

    pltpu.sync_copy(src_hbm.at[wid], src_v)
    pltpu.sync_copy(dst_hbm.at[wid], dst_v)
    plsc.subcore_barrier()

    def body(j, carry):
        pltpu.async_copy(x_hbm.at[src_v.at[j]], rows_v, sem).wait()
        pltpu.sync_copy(rows_v, acc.at[dst_v.at[j]], add=True)
        return carry

    lax.fori_loop(0, NCH, body, 0)
    plsc.subcore_barrier()

    @pl.when(s < 15)
    def _():
        pltpu.sync_copy(acc.at[pl.ds(off, RPT)],
                        out_hbm.at[c, pl.ds(off, RPT)])

    @pl.when(s == 15)
    def _():
        pltpu.sync_copy(acc.at[pl.ds(15 * RPT, RPT_LAST)],
                        out_hbm.at[c, pl.ds(15 * RPT, RPT_LAST)])


def _mlp_body(p_ref, wa_ref, ba_ref, wb_ref, bb_ref, o_ref):
    z = p_ref[0] + p_ref[1]
    h = jnp.dot(z, wa_ref[...], preferred_element_type=jnp.float32,
                precision=lax.Precision.HIGHEST) + ba_ref[...]
    h = jnp.maximum(h, 0.0)
    h = jnp.dot(h, wb_ref[...], preferred_element_type=jnp.float32,
                precision=lax.Precision.HIGHEST) + bb_ref[...]
    o_ref[...] = jnp.maximum(h, 0.0)


_mlp_call = pl.pallas_call(
    _mlp_body,
    out_shape=jax.ShapeDtypeStruct((N, H), jnp.float32),
)


def _head_body(p_ref, wa_ref, ba_ref, wb_ref, bb_ref, batch_ref, wf_ref,
               bf_ref, o_ref):
    z = p_ref[0] + p_ref[1]
    h = jnp.dot(z, wa_ref[...], preferred_element_type=jnp.float32,
                precision=lax.Precision.HIGHEST) + ba_ref[...]
    h = jnp.maximum(h, 0.0)
    h = jnp.dot(h, wb_ref[...], preferred_element_type=jnp.float32,
                precision=lax.Precision.HIGHEST) + bb_ref[...]
    h = jnp.maximum(h, 0.0)
    gids = lax.broadcasted_iota(jnp.int32, (G, 1), 0)
    mask = (gids == batch_ref[...]).astype(jnp.float32)
    pooled = jnp.dot(mask, h, preferred_element_type=jnp.float32,
                     precision=lax.Precision.HIGHEST)
    logits = jnp.dot(pooled, wf_ref[...], preferred_element_type=jnp.float32,
                     precision=lax.Precision.HIGHEST) + bf_ref[...]
    m = jnp.max(logits, axis=-1, keepdims=True)
    e = jnp.exp(logits - m)
    lse = jnp.log(jnp.sum(e, axis=-1, keepdims=True)) + m
    o_ref[...] = logits - lse


_head_call = pl.pallas_call(
    _head_body,
    out_shape=jax.ShapeDtypeStruct((G, OUT), jnp.float32),
)


def kernel(x, edge_index, batch, W1a, b1a, W1b, b1b, W2a, b2a, W2b, b2b,
           Wf, bf):
    src = edge_index[0].reshape(NTILES, NCH, CHUNK)
    dst = edge_index[1].reshape(NTILES, NCH, CHUNK)
    zeros = jnp.zeros((RPT, D), jnp.float32)

    p1 = _scatter_agg(x, src, dst, zeros)
    h1 = _mlp_call(p1, W1a, b1a.reshape(1, H), W1b, b1b.reshape(1, H))
    p2 = _scatter_agg(h1, src, dst, zeros)
    return _head_call(p2, W2a, b2a.reshape(1, H), W2b, b2b.reshape(1, H),
                      batch.reshape(1, N), Wf, bf.reshape(1, OUT))


# speedup vs baseline: 9.1672x; 1.4211x over previous
import functools

import jax
import jax.numpy as jnp
from jax import lax
from jax.experimental import pallas as pl
from jax.experimental.pallas import tpu as pltpu
from jax.experimental.pallas import tpu_sc as plsc

N = 10000
E = 320000
D = 128
H = 128
OUT = 10
G = 64

NTILES = 32
EPT = E // NTILES
CHUNK = 40
NCH = EPT // CHUNK
NBUF = 5
NIDX = 5
AHEAD = 4
RPT = 632
RPT_LAST = N - 15 * RPT

_mesh = plsc.VectorSubcoreMesh(core_axis_name="c", subcore_axis_name="s")


@functools.partial(
    pl.kernel,
    out_type=jax.ShapeDtypeStruct((2, N, D), jnp.float32),
    mesh=_mesh,
    scratch_types=[
        pltpu.VMEM((NIDX, 2, CHUNK), jnp.int32),
        pltpu.VMEM((CHUNK, D), jnp.float32),
        pltpu.VMEM((CHUNK, D), jnp.float32),
        pltpu.VMEM((CHUNK, D), jnp.float32),
        pltpu.VMEM((CHUNK, D), jnp.float32),
        pltpu.VMEM((CHUNK, D), jnp.float32),
        pltpu.VMEM_SHARED((N, D), jnp.float32),
        pltpu.SemaphoreType.DMA((NIDX,)),
        pltpu.SemaphoreType.DMA((NBUF,)),
    ],
)
def _scatter_agg(x_hbm, idx_hbm, zeros_hbm, out_hbm,
                 idx_v, r0, r1, r2, r3, r4, acc, isem, gsem):
    rows = [r0, r1, r2, r3, r4]
    c = lax.axis_index("c")
    s = lax.axis_index("s")
    wid = c * 16 + s

    def _iwait(sl):
        pltpu.make_async_copy(idx_hbm.at[0, 0], idx_v.at[sl],
                              isem.at[sl]).wait()

    def _gwait(buf):
        pltpu.make_async_copy(x_hbm.at[pl.ds(0, CHUNK)],
                              rows[buf], gsem.at[buf]).wait()

    for k in range(AHEAD + 1):
        pltpu.async_copy(idx_hbm.at[wid, k], idx_v.at[k % NIDX],
                         isem.at[k % NIDX])

    off = pl.multiple_of(s * RPT, 8)

    @pl.when((c == 0) & (s < 15))
    def _():
        pltpu.sync_copy(x_hbm.at[pl.ds(off, RPT)], acc.at[pl.ds(off, RPT)])

    @pl.when((c == 0) & (s == 15))
    def _():
        pltpu.sync_copy(x_hbm.at[pl.ds(15 * RPT, RPT_LAST)],
                        acc.at[pl.ds(15 * RPT, RPT_LAST)])

    @pl.when((c != 0) & (s < 15))
    def _():
        pltpu.sync_copy(zeros_hbm.at[pl.ds(0, RPT)], acc.at[pl.ds(off, RPT)])

    @pl.when((c != 0) & (s == 15))
    def _():
        pltpu.sync_copy(zeros_hbm.at[pl.ds(0, RPT_LAST)],
                        acc.at[pl.ds(15 * RPT, RPT_LAST)])

    for k in range(AHEAD):
        _iwait(k % NIDX)
        pltpu.async_copy(x_hbm.at[idx_v.at[k % NIDX, 0]], rows[k % NBUF],
                         gsem.at[k % NBUF])

    plsc.subcore_barrier()

    def body(o, carry):
        for b0 in range(NBUF):
            j = o * NBUF + b0
            _gwait(b0)
            pltpu.sync_copy(rows[b0], acc.at[idx_v.at[b0 % NIDX, 1]],
                            add=True)
            jp = j + AHEAD + 1
            sp = (b0 + AHEAD + 1) % NIDX

            @pl.when(jp < NCH)
            def _():
                pltpu.async_copy(idx_hbm.at[wid, jp], idx_v.at[sp],
                                 isem.at[sp])

            jg = j + AHEAD
            sg = (b0 + AHEAD) % NIDX
            rg = (b0 + AHEAD) % NBUF

            @pl.when(jg < NCH)
            def _():
                _iwait(sg)
                pltpu.async_copy(x_hbm.at[idx_v.at[sg, 0]], rows[rg],
                                 gsem.at[rg])
        return carry

    lax.fori_loop(0, NCH // NBUF, body, 0)
    plsc.subcore_barrier()

    @pl.when(s < 15)
    def _():
        pltpu.sync_copy(acc.at[pl.ds(off, RPT)],
                        out_hbm.at[c, pl.ds(off, RPT)])

    @pl.when(s == 15)
    def _():
        pltpu.sync_copy(acc.at[pl.ds(15 * RPT, RPT_LAST)],
                        out_hbm.at[c, pl.ds(15 * RPT, RPT_LAST)])


def _mlp_body(p_ref, wa_ref, ba_ref, wb_ref, bb_ref, o_ref):
    z = p_ref[0] + p_ref[1]
    h = jnp.dot(z, wa_ref[...], preferred_element_type=jnp.float32,
                precision=lax.Precision.HIGHEST) + ba_ref[...]
    h = jnp.maximum(h, 0.0)
    h = jnp.dot(h, wb_ref[...], preferred_element_type=jnp.float32,
                precision=lax.Precision.HIGHEST) + bb_ref[...]
    o_ref[...] = jnp.maximum(h, 0.0)


_mlp_call = pl.pallas_call(
    _mlp_body,
    out_shape=jax.ShapeDtypeStruct((N, H), jnp.float32),
)


def _head_body(p_ref, wa_ref, ba_ref, wb_ref, bb_ref, batch_ref, wf_ref,
               bf_ref, o_ref):
    z = p_ref[0] + p_ref[1]
    h = jnp.dot(z, wa_ref[...], preferred_element_type=jnp.float32,
                precision=lax.Precision.HIGHEST) + ba_ref[...]
    h = jnp.maximum(h, 0.0)
    h = jnp.dot(h, wb_ref[...], preferred_element_type=jnp.float32,
                precision=lax.Precision.HIGHEST) + bb_ref[...]
    h = jnp.maximum(h, 0.0)
    gids = lax.broadcasted_iota(jnp.int32, (G, 1), 0)
    mask = (gids == batch_ref[...]).astype(jnp.float32)
    pooled = jnp.dot(mask, h, preferred_element_type=jnp.float32,
                     precision=lax.Precision.HIGHEST)
    logits = jnp.dot(pooled, wf_ref[...], preferred_element_type=jnp.float32,
                     precision=lax.Precision.HIGHEST) + bf_ref[...]
    m = jnp.max(logits, axis=-1, keepdims=True)
    e = jnp.exp(logits - m)
    lse = jnp.log(jnp.sum(e, axis=-1, keepdims=True)) + m
    o_ref[...] = logits - lse


_head_call = pl.pallas_call(
    _head_body,
    out_shape=jax.ShapeDtypeStruct((G, OUT), jnp.float32),
)


def kernel(x, edge_index, batch, W1a, b1a, W1b, b1b, W2a, b2a, W2b, b2b,
           Wf, bf):
    src = edge_index[0].reshape(NTILES, NCH, 1, CHUNK)
    dst = edge_index[1].reshape(NTILES, NCH, 1, CHUNK)
    idx = jnp.concatenate([src, dst], axis=2)
    zeros = jnp.zeros((RPT, D), jnp.float32)

    p1 = _scatter_agg(x, idx, zeros)
    h1 = _mlp_call(p1, W1a, b1a.reshape(1, H), W1b, b1b.reshape(1, H))
    p2 = _scatter_agg(h1, idx, zeros)
    return _head_call(p2, W2a, b2a.reshape(1, H), W2b, b2b.reshape(1, H),
                      batch.reshape(1, N), Wf, bf.reshape(1, OUT))


# speedup vs baseline: 10.1985x; 1.1125x over previous
import functools

import jax
import jax.numpy as jnp
from jax import lax
from jax.experimental import pallas as pl
from jax.experimental.pallas import tpu as pltpu
from jax.experimental.pallas import tpu_sc as plsc

N = 10000
E = 320000
D = 128
H = 128
OUT = 10
G = 64

NTILES = 32
EPT = E // NTILES
CHUNK = 40
NCH = EPT // CHUNK
NBUF = 5
NIDX = 5
AHEAD = 4
RPT = 632
RPT_LAST = N - 15 * RPT

_mesh = plsc.VectorSubcoreMesh(core_axis_name="c", subcore_axis_name="s")


@functools.partial(
    pl.kernel,
    out_type=jax.ShapeDtypeStruct((2, N, D), jnp.float32),
    mesh=_mesh,
    scratch_types=[
        pltpu.VMEM((NIDX, 2, CHUNK), jnp.int32),
        pltpu.VMEM((CHUNK, D), jnp.float32),
        pltpu.VMEM((CHUNK, D), jnp.float32),
        pltpu.VMEM((CHUNK, D), jnp.float32),
        pltpu.VMEM((CHUNK, D), jnp.float32),
        pltpu.VMEM((CHUNK, D), jnp.float32),
        pltpu.VMEM_SHARED((N, D), jnp.float32),
        pltpu.SemaphoreType.DMA((NIDX,)),
        pltpu.SemaphoreType.DMA((NBUF,)),
    ],
)
def _scatter_agg(x_hbm, idx_hbm, zeros_hbm, out_hbm,
                 idx_v, r0, r1, r2, r3, r4, acc, isem, gsem):
    rows = [r0, r1, r2, r3, r4]
    c = lax.axis_index("c")
    s = lax.axis_index("s")
    wid = c * 16 + s

    def _iwait(sl):
        pltpu.make_async_copy(idx_hbm.at[0, 0], idx_v.at[sl],
                              isem.at[sl]).wait()

    def _gwait(buf):
        pltpu.make_async_copy(x_hbm.at[pl.ds(0, CHUNK)],
                              rows[buf], gsem.at[buf]).wait()

    for k in range(AHEAD + 1):
        pltpu.async_copy(idx_hbm.at[wid, k], idx_v.at[k % NIDX],
                         isem.at[k % NIDX])

    off = pl.multiple_of(s * RPT, 8)

    @pl.when((c == 0) & (s < 15))
    def _():
        pltpu.sync_copy(x_hbm.at[pl.ds(off, RPT)], acc.at[pl.ds(off, RPT)])

    @pl.when((c == 0) & (s == 15))
    def _():
        pltpu.sync_copy(x_hbm.at[pl.ds(15 * RPT, RPT_LAST)],
                        acc.at[pl.ds(15 * RPT, RPT_LAST)])

    @pl.when((c != 0) & (s < 15))
    def _():
        pltpu.sync_copy(zeros_hbm.at[pl.ds(0, RPT)], acc.at[pl.ds(off, RPT)])

    @pl.when((c != 0) & (s == 15))
    def _():
        pltpu.sync_copy(zeros_hbm.at[pl.ds(0, RPT_LAST)],
                        acc.at[pl.ds(15 * RPT, RPT_LAST)])

    for k in range(AHEAD):
        _iwait(k % NIDX)
        pltpu.async_copy(x_hbm.at[idx_v.at[k % NIDX, 0]], rows[k % NBUF],
                         gsem.at[k % NBUF])

    plsc.subcore_barrier()

    def body(o, carry):
        for b0 in range(NBUF):
            j = o * NBUF + b0
            _gwait(b0)
            pltpu.sync_copy(rows[b0], acc.at[idx_v.at[b0 % NIDX, 1]],
                            add=True)
            jp = j + AHEAD + 1
            sp = (b0 + AHEAD + 1) % NIDX

            @pl.when(jp < NCH)
            def _():
                pltpu.async_copy(idx_hbm.at[wid, jp], idx_v.at[sp],
                                 isem.at[sp])

            jg = j + AHEAD
            sg = (b0 + AHEAD) % NIDX
            rg = (b0 + AHEAD) % NBUF

            @pl.when(jg < NCH)
            def _():
                _iwait(sg)
                pltpu.async_copy(x_hbm.at[idx_v.at[sg, 0]], rows[rg],
                                 gsem.at[rg])
        return carry

    lax.fori_loop(0, NCH // NBUF, body, 0)
    plsc.subcore_barrier()

    @pl.when(s < 15)
    def _():
        pltpu.sync_copy(acc.at[pl.ds(off, RPT)],
                        out_hbm.at[c, pl.ds(off, RPT)])

    @pl.when(s == 15)
    def _():
        pltpu.sync_copy(acc.at[pl.ds(15 * RPT, RPT_LAST)],
                        out_hbm.at[c, pl.ds(15 * RPT, RPT_LAST)])


def _mlp_body(p_ref, wa_ref, ba_ref, wb_ref, bb_ref, o_ref):
    z = p_ref[0] + p_ref[1]
    h = jnp.dot(z, wa_ref[...], preferred_element_type=jnp.float32,
                precision=lax.Precision.DEFAULT) + ba_ref[...]
    h = jnp.maximum(h, 0.0)
    h = jnp.dot(h, wb_ref[...], preferred_element_type=jnp.float32,
                precision=lax.Precision.DEFAULT) + bb_ref[...]
    o_ref[...] = jnp.maximum(h, 0.0)


_mlp_call = pl.pallas_call(
    _mlp_body,
    out_shape=jax.ShapeDtypeStruct((N, H), jnp.float32),
)


def _head_body(p_ref, wa_ref, ba_ref, wb_ref, bb_ref, batch_ref, wf_ref,
               bf_ref, o_ref):
    z = p_ref[0] + p_ref[1]
    h = jnp.dot(z, wa_ref[...], preferred_element_type=jnp.float32,
                precision=lax.Precision.DEFAULT) + ba_ref[...]
    h = jnp.maximum(h, 0.0)
    h = jnp.dot(h, wb_ref[...], preferred_element_type=jnp.float32,
                precision=lax.Precision.DEFAULT) + bb_ref[...]
    h = jnp.maximum(h, 0.0)
    gids = lax.broadcasted_iota(jnp.int32, (G, 1), 0)
    mask = (gids == batch_ref[...]).astype(jnp.float32)
    pooled = jnp.dot(mask, h, preferred_element_type=jnp.float32,
                     precision=lax.Precision.DEFAULT)
    logits = jnp.dot(pooled, wf_ref[...], preferred_element_type=jnp.float32,
                     precision=lax.Precision.DEFAULT) + bf_ref[...]
    m = jnp.max(logits, axis=-1, keepdims=True)
    e = jnp.exp(logits - m)
    lse = jnp.log(jnp.sum(e, axis=-1, keepdims=True)) + m
    o_ref[...] = logits - lse


_head_call = pl.pallas_call(
    _head_body,
    out_shape=jax.ShapeDtypeStruct((G, OUT), jnp.float32),
)


def kernel(x, edge_index, batch, W1a, b1a, W1b, b1b, W2a, b2a, W2b, b2b,
           Wf, bf):
    src = edge_index[0].reshape(NTILES, NCH, 1, CHUNK)
    dst = edge_index[1].reshape(NTILES, NCH, 1, CHUNK)
    idx = jnp.concatenate([src, dst], axis=2)
    zeros = jnp.zeros((RPT, D), jnp.float32)

    p1 = _scatter_agg(x, idx, zeros)
    h1 = _mlp_call(p1, W1a, b1a.reshape(1, H), W1b, b1b.reshape(1, H))
    p2 = _scatter_agg(h1, idx, zeros)
    return _head_call(p2, W2a, b2a.reshape(1, H), W2b, b2b.reshape(1, H),
                      batch.reshape(1, N), Wf, bf.reshape(1, OUT))


# speedup vs baseline: 10.9518x; 1.0739x over previous
import functools

import jax
import jax.numpy as jnp
from jax import lax
from jax.experimental import pallas as pl
from jax.experimental.pallas import tpu as pltpu
from jax.experimental.pallas import tpu_sc as plsc

N = 10000
E = 320000
D = 128
H = 128
OUT = 10
G = 64

NTILES = 32
EPT = E // NTILES
CHUNK = 80
NCH = EPT // CHUNK
NBUF = 3
NIDX = 3
AHEAD = 2
NMAIN = (NCH // NBUF) * NBUF
NPEEL = NCH - NMAIN
RPT = 632
RPT_LAST = N - 15 * RPT

_mesh = plsc.VectorSubcoreMesh(core_axis_name="c", subcore_axis_name="s")


@functools.partial(
    pl.kernel,
    out_type=jax.ShapeDtypeStruct((2, N, D), jnp.float32),
    mesh=_mesh,
    scratch_types=[
        pltpu.VMEM((NIDX, 2, CHUNK), jnp.int32),
        pltpu.VMEM((CHUNK, D), jnp.float32),
        pltpu.VMEM((CHUNK, D), jnp.float32),
        pltpu.VMEM((CHUNK, D), jnp.float32),
        pltpu.VMEM_SHARED((N, D), jnp.float32),
        pltpu.SemaphoreType.DMA((NIDX,)),
        pltpu.SemaphoreType.DMA((NBUF,)),
    ],
)
def _scatter_agg(x_hbm, idx_hbm, zeros_hbm, out_hbm,
                 idx_v, r0, r1, r2, acc, isem, gsem):
    rows = [r0, r1, r2]
    c = lax.axis_index("c")
    s = lax.axis_index("s")
    wid = c * 16 + s

    def _iwait(sl):
        pltpu.make_async_copy(idx_hbm.at[0, 0], idx_v.at[sl],
                              isem.at[sl]).wait()

    def _gwait(buf):
        pltpu.make_async_copy(x_hbm.at[pl.ds(0, CHUNK)],
                              rows[buf], gsem.at[buf]).wait()

    for k in range(AHEAD + 1):
        pltpu.async_copy(idx_hbm.at[wid, k], idx_v.at[k % NIDX],
                         isem.at[k % NIDX])

    off = pl.multiple_of(s * RPT, 8)

    @pl.when((c == 0) & (s < 15))
    def _():
        pltpu.sync_copy(x_hbm.at[pl.ds(off, RPT)], acc.at[pl.ds(off, RPT)])

    @pl.when((c == 0) & (s == 15))
    def _():
        pltpu.sync_copy(x_hbm.at[pl.ds(15 * RPT, RPT_LAST)],
                        acc.at[pl.ds(15 * RPT, RPT_LAST)])

    @pl.when((c != 0) & (s < 15))
    def _():
        pltpu.sync_copy(zeros_hbm.at[pl.ds(0, RPT)], acc.at[pl.ds(off, RPT)])

    @pl.when((c != 0) & (s == 15))
    def _():
        pltpu.sync_copy(zeros_hbm.at[pl.ds(0, RPT_LAST)],
                        acc.at[pl.ds(15 * RPT, RPT_LAST)])

    for k in range(AHEAD):
        _iwait(k % NIDX)
        pltpu.async_copy(x_hbm.at[idx_v.at[k % NIDX, 0]], rows[k % NBUF],
                         gsem.at[k % NBUF])

    plsc.subcore_barrier()

    def body(o, carry):
        for b0 in range(NBUF):
            j = o * NBUF + b0
            _gwait(b0)
            pltpu.sync_copy(rows[b0], acc.at[idx_v.at[b0 % NIDX, 1]],
                            add=True)
            jp = j + AHEAD + 1
            sp = (b0 + AHEAD + 1) % NIDX

            @pl.when(jp < NCH)
            def _():
                pltpu.async_copy(idx_hbm.at[wid, jp], idx_v.at[sp],
                                 isem.at[sp])

            jg = j + AHEAD
            sg = (b0 + AHEAD) % NIDX
            rg = (b0 + AHEAD) % NBUF

            @pl.when(jg < NCH)
            def _():
                _iwait(sg)
                pltpu.async_copy(x_hbm.at[idx_v.at[sg, 0]], rows[rg],
                                 gsem.at[rg])
        return carry

    lax.fori_loop(0, NMAIN // NBUF, body, 0)
    for t in range(NPEEL):
        j = NMAIN + t
        b = j % NBUF
        _gwait(b)
        pltpu.sync_copy(rows[b], acc.at[idx_v.at[j % NIDX, 1]], add=True)
    plsc.subcore_barrier()

    @pl.when(s < 15)
    def _():
        pltpu.sync_copy(acc.at[pl.ds(off, RPT)],
                        out_hbm.at[c, pl.ds(off, RPT)])

    @pl.when(s == 15)
    def _():
        pltpu.sync_copy(acc.at[pl.ds(15 * RPT, RPT_LAST)],
                        out_hbm.at[c, pl.ds(15 * RPT, RPT_LAST)])


def _mlp_body(p_ref, wa_ref, ba_ref, wb_ref, bb_ref, o_ref):
    z = p_ref[0] + p_ref[1]
    h = jnp.dot(z, wa_ref[...], preferred_element_type=jnp.float32,
                precision=lax.Precision.DEFAULT) + ba_ref[...]
    h = jnp.maximum(h, 0.0)
    h = jnp.dot(h, wb_ref[...], preferred_element_type=jnp.float32,
                precision=lax.Precision.DEFAULT) + bb_ref[...]
    o_ref[...] = jnp.maximum(h, 0.0)


_mlp_call = pl.pallas_call(
    _mlp_body,
    out_shape=jax.ShapeDtypeStruct((N, H), jnp.float32),
)


def _head_body(p_ref, wa_ref, ba_ref, wb_ref, bb_ref, batch_ref, wf_ref,
               bf_ref, o_ref):
    z = p_ref[0] + p_ref[1]
    h = jnp.dot(z, wa_ref[...], preferred_element_type=jnp.float32,
                precision=lax.Precision.DEFAULT) + ba_ref[...]
    h = jnp.maximum(h, 0.0)
    h = jnp.dot(h, wb_ref[...], preferred_element_type=jnp.float32,
                precision=lax.Precision.DEFAULT) + bb_ref[...]
    h = jnp.maximum(h, 0.0)
    gids = lax.broadcasted_iota(jnp.int32, (G, 1), 0)
    mask = (gids == batch_ref[...]).astype(jnp.float32)
    pooled = jnp.dot(mask, h, preferred_element_type=jnp.float32,
                     precision=lax.Precision.DEFAULT)
    logits = jnp.dot(pooled, wf_ref[...], preferred_element_type=jnp.float32,
                     precision=lax.Precision.DEFAULT) + bf_ref[...]
    m = jnp.max(logits, axis=-1, keepdims=True)
    e = jnp.exp(logits - m)
    lse = jnp.log(jnp.sum(e, axis=-1, keepdims=True)) + m
    o_ref[...] = logits - lse


_head_call = pl.pallas_call(
    _head_body,
    out_shape=jax.ShapeDtypeStruct((G, OUT), jnp.float32),
)


def kernel(x, edge_index, batch, W1a, b1a, W1b, b1b, W2a, b2a, W2b, b2b,
           Wf, bf):
    src = edge_index[0].reshape(NTILES, NCH, 1, CHUNK)
    dst = edge_index[1].reshape(NTILES, NCH, 1, CHUNK)
    idx = jnp.concatenate([src, dst], axis=2)
    zeros = jnp.zeros((RPT, D), jnp.float32)

    p1 = _scatter_agg(x, idx, zeros)
    h1 = _mlp_call(p1, W1a, b1a.reshape(1, H), W1b, b1b.reshape(1, H))
    p2 = _scatter_agg(h1, idx, zeros)
    return _head_call(p2, W2a, b2a.reshape(1, H), W2b, b2b.reshape(1, H),
                      batch.reshape(1, N), Wf, bf.reshape(1, OUT))


# speedup vs baseline: 12.3026x; 1.1233x over previous
import functools

import jax
import jax.numpy as jnp
from jax import lax
from jax.experimental import pallas as pl
from jax.experimental.pallas import tpu as pltpu
from jax.experimental.pallas import tpu_sc as plsc

N = 10000
E = 320000
D = 128
H = 128
OUT = 10
G = 64

NTILES = 32
EPT = E // NTILES
CHUNK = 80
NCH = EPT // CHUNK
NBUF = 3
NIDX = 6
AHEAD = 2
FETCH = 3
UNROLL = 6
NMAIN = (NCH // UNROLL) * UNROLL
NPEEL = NCH - NMAIN
RPT = 632
RPT_LAST = N - 15 * RPT

_mesh = plsc.VectorSubcoreMesh(core_axis_name="c", subcore_axis_name="s")


@functools.partial(
    pl.kernel,
    out_type=jax.ShapeDtypeStruct((2, N, D), jnp.float32),
    mesh=_mesh,
    scratch_types=[
        pltpu.VMEM((NIDX, 2, CHUNK), jnp.int32),
        pltpu.VMEM((CHUNK, D), jnp.float32),
        pltpu.VMEM((CHUNK, D), jnp.float32),
        pltpu.VMEM((CHUNK, D), jnp.float32),
        pltpu.VMEM_SHARED((N, D), jnp.float32),
        pltpu.SemaphoreType.DMA((NIDX,)),
        pltpu.SemaphoreType.DMA((NBUF,)),
        pltpu.SemaphoreType.DMA((NBUF,)),
    ],
)
def _scatter_agg(x_hbm, idx_hbm, zeros_hbm, out_hbm,
                 idx_v, r0, r1, r2, acc, isem, gsem, ssem):
    rows = [r0, r1, r2]
    c = lax.axis_index("c")
    s = lax.axis_index("s")
    wid = c * 16 + s

    def _iwait(sl):
        pltpu.make_async_copy(idx_hbm.at[0, 0], idx_v.at[sl],
                              isem.at[sl]).wait()

    def _gwait(buf):
        pltpu.make_async_copy(x_hbm.at[pl.ds(0, CHUNK)],
                              rows[buf], gsem.at[buf]).wait()

    def _swait(buf):
        pltpu.make_async_copy(x_hbm.at[pl.ds(0, CHUNK)],
                              rows[buf], ssem.at[buf]).wait()

    def _step(u, j, swait_ok, fetch_ok, gather_ok):
        b0 = u % NBUF
        _gwait(b0)
        pltpu.async_copy(rows[b0], acc.at[idx_v.at[u % NIDX, 1]],
                         ssem.at[b0], add=True)
        if fetch_ok:
            sp = (u + FETCH) % NIDX
            pltpu.async_copy(idx_hbm.at[wid, j + FETCH], idx_v.at[sp],
                             isem.at[sp])
        if gather_ok:
            rg = (u + AHEAD) % NBUF
            sg = (u + AHEAD) % NIDX
            if swait_ok:
                _swait(rg)
            _iwait(sg)
            pltpu.async_copy(x_hbm.at[idx_v.at[sg, 0]], rows[rg],
                             gsem.at[rg])

    for k in range(FETCH):
        pltpu.async_copy(idx_hbm.at[wid, k], idx_v.at[k % NIDX],
                         isem.at[k % NIDX])

    off = pl.multiple_of(s * RPT, 8)

    @pl.when((c == 0) & (s < 15))
    def _():
        pltpu.sync_copy(x_hbm.at[pl.ds(off, RPT)], acc.at[pl.ds(off, RPT)])

    @pl.when((c == 0) & (s == 15))
    def _():
        pltpu.sync_copy(x_hbm.at[pl.ds(15 * RPT, RPT_LAST)],
                        acc.at[pl.ds(15 * RPT, RPT_LAST)])

    @pl.when((c != 0) & (s < 15))
    def _():
        pltpu.sync_copy(zeros_hbm.at[pl.ds(0, RPT)], acc.at[pl.ds(off, RPT)])

    @pl.when((c != 0) & (s == 15))
    def _():
        pltpu.sync_copy(zeros_hbm.at[pl.ds(0, RPT_LAST)],
                        acc.at[pl.ds(15 * RPT, RPT_LAST)])

    for k in range(AHEAD):
        _iwait(k % NIDX)
        pltpu.async_copy(x_hbm.at[idx_v.at[k % NIDX, 0]], rows[k % NBUF],
                         gsem.at[k % NBUF])

    plsc.subcore_barrier()

    for u in range(UNROLL):
        _step(u, u, swait_ok=(u >= 1), fetch_ok=True, gather_ok=True)

    def body(o, carry):
        for u in range(UNROLL):
            _step(u, o * UNROLL + u, swait_ok=True, fetch_ok=True,
                  gather_ok=True)
        return carry

    lax.fori_loop(1, NMAIN // UNROLL, body, 0)
    for t in range(NPEEL):
        j = NMAIN + t
        _step(t, j, swait_ok=True, fetch_ok=(j + FETCH < NCH),
              gather_ok=(j + AHEAD < NCH))
    for b in range(NBUF):
        _swait(b)
    plsc.subcore_barrier()

    @pl.when(s < 15)
    def _():
        pltpu.sync_copy(acc.at[pl.ds(off, RPT)],
                        out_hbm.at[c, pl.ds(off, RPT)])

    @pl.when(s == 15)
    def _():
        pltpu.sync_copy(acc.at[pl.ds(15 * RPT, RPT_LAST)],
                        out_hbm.at[c, pl.ds(15 * RPT, RPT_LAST)])


def _mlp_body(p_ref, wa_ref, ba_ref, wb_ref, bb_ref, o_ref):
    z = p_ref[0] + p_ref[1]
    h = jnp.dot(z, wa_ref[...], preferred_element_type=jnp.float32,
                precision=lax.Precision.DEFAULT) + ba_ref[...]
    h = jnp.maximum(h, 0.0)
    h = jnp.dot(h, wb_ref[...], preferred_element_type=jnp.float32,
                precision=lax.Precision.DEFAULT) + bb_ref[...]
    o_ref[...] = jnp.maximum(h, 0.0)


_mlp_call = pl.pallas_call(
    _mlp_body,
    out_shape=jax.ShapeDtypeStruct((N, H), jnp.float32),
)


def _head_body(p_ref, wa_ref, ba_ref, wb_ref, bb_ref, batch_ref, wf_ref,
               bf_ref, o_ref):
    z = p_ref[0] + p_ref[1]
    h = jnp.dot(z, wa_ref[...], preferred_element_type=jnp.float32,
                precision=lax.Precision.DEFAULT) + ba_ref[...]
    h = jnp.maximum(h, 0.0)
    h = jnp.dot(h, wb_ref[...], preferred_element_type=jnp.float32,
                precision=lax.Precision.DEFAULT) + bb_ref[...]
    h = jnp.maximum(h, 0.0)
    gids = lax.broadcasted_iota(jnp.int32, (G, 1), 0)
    mask = (gids == batch_ref[...]).astype(jnp.float32)
    pooled = jnp.dot(mask, h, preferred_element_type=jnp.float32,
                     precision=lax.Precision.DEFAULT)
    logits = jnp.dot(pooled, wf_ref[...], preferred_element_type=jnp.float32,
                     precision=lax.Precision.DEFAULT) + bf_ref[...]
    m = jnp.max(logits, axis=-1, keepdims=True)
    e = jnp.exp(logits - m)
    lse = jnp.log(jnp.sum(e, axis=-1, keepdims=True)) + m
    o_ref[...] = logits - lse


_head_call = pl.pallas_call(
    _head_body,
    out_shape=jax.ShapeDtypeStruct((G, OUT), jnp.float32),
)


def kernel(x, edge_index, batch, W1a, b1a, W1b, b1b, W2a, b2a, W2b, b2b,
           Wf, bf):
    src = edge_index[0].reshape(NTILES, NCH, 1, CHUNK)
    dst = edge_index[1].reshape(NTILES, NCH, 1, CHUNK)
    idx = jnp.concatenate([src, dst], axis=2)
    zeros = jnp.zeros((RPT, D), jnp.float32)

    p1 = _scatter_agg(x, idx, zeros)
    h1 = _mlp_call(p1, W1a, b1a.reshape(1, H), W1b, b1b.reshape(1, H))
    p2 = _scatter_agg(h1, idx, zeros)
    return _head_call(p2, W2a, b2a.reshape(1, H), W2b, b2b.reshape(1, H),
                      batch.reshape(1, N), Wf, bf.reshape(1, OUT))


# speedup vs baseline: 12.3169x; 1.0012x over previous
import functools

import jax
import jax.numpy as jnp
from jax import lax
from jax.experimental import pallas as pl
from jax.experimental.pallas import tpu as pltpu
from jax.experimental.pallas import tpu_sc as plsc

N = 10000
E = 320000
D = 128
H = 128
OUT = 10
G = 64

NTILES = 32
EPT = E // NTILES
CHUNK = 80
NCH = EPT // CHUNK
NBUF = 4
NIDX = 8
AHEAD = 3
FETCH = 4
UNROLL = 8
NMAIN = (NCH // UNROLL) * UNROLL
NPEEL = NCH - NMAIN
RPT = 632
RPT_LAST = N - 15 * RPT

_mesh = plsc.VectorSubcoreMesh(core_axis_name="c", subcore_axis_name="s")


@functools.partial(
    pl.kernel,
    out_type=jax.ShapeDtypeStruct((2, N, D), jnp.float32),
    mesh=_mesh,
    scratch_types=[
        pltpu.VMEM((NIDX, 2, CHUNK), jnp.int32),
        pltpu.VMEM((CHUNK, D), jnp.float32),
        pltpu.VMEM((CHUNK, D), jnp.float32),
        pltpu.VMEM((CHUNK, D), jnp.float32),
        pltpu.VMEM((CHUNK, D), jnp.float32),
        pltpu.VMEM_SHARED((N, D), jnp.float32),
        pltpu.SemaphoreType.DMA((NIDX,)),
        pltpu.SemaphoreType.DMA((NBUF,)),
        pltpu.SemaphoreType.DMA((NBUF,)),
    ],
)
def _scatter_agg(x_hbm, idx_hbm, zeros_hbm, out_hbm,
                 idx_v, r0, r1, r2, r3, acc, isem, gsem, ssem):
    rows = [r0, r1, r2, r3]
    c = lax.axis_index("c")
    s = lax.axis_index("s")
    wid = c * 16 + s

    def _iwait(sl):
        pltpu.make_async_copy(idx_hbm.at[0, 0], idx_v.at[sl],
                              isem.at[sl]).wait()

    def _gwait(buf):
        pltpu.make_async_copy(x_hbm.at[pl.ds(0, CHUNK)],
                              rows[buf], gsem.at[buf]).wait()

    def _swait(buf):
        pltpu.make_async_copy(x_hbm.at[pl.ds(0, CHUNK)],
                              rows[buf], ssem.at[buf]).wait()

    def _step(u, j, swait_ok, fetch_ok, gather_ok):
        b0 = u % NBUF
        _gwait(b0)
        pltpu.async_copy(rows[b0], acc.at[idx_v.at[u % NIDX, 1]],
                         ssem.at[b0], add=True)
        if fetch_ok:
            sp = (u + FETCH) % NIDX
            pltpu.async_copy(idx_hbm.at[wid, j + FETCH], idx_v.at[sp],
                             isem.at[sp])
        if gather_ok:
            rg = (u + AHEAD) % NBUF
            sg = (u + AHEAD) % NIDX
            if swait_ok:
                _swait(rg)
            _iwait(sg)
            pltpu.async_copy(x_hbm.at[idx_v.at[sg, 0]], rows[rg],
                             gsem.at[rg])

    for k in range(FETCH):
        pltpu.async_copy(idx_hbm.at[wid, k], idx_v.at[k % NIDX],
                         isem.at[k % NIDX])

    off = pl.multiple_of(s * RPT, 8)

    @pl.when((c == 0) & (s < 15))
    def _():
        pltpu.sync_copy(x_hbm.at[pl.ds(off, RPT)], acc.at[pl.ds(off, RPT)])

    @pl.when((c == 0) & (s == 15))
    def _():
        pltpu.sync_copy(x_hbm.at[pl.ds(15 * RPT, RPT_LAST)],
                        acc.at[pl.ds(15 * RPT, RPT_LAST)])

    @pl.when((c != 0) & (s < 15))
    def _():
        pltpu.sync_copy(zeros_hbm.at[pl.ds(0, RPT)], acc.at[pl.ds(off, RPT)])

    @pl.when((c != 0) & (s == 15))
    def _():
        pltpu.sync_copy(zeros_hbm.at[pl.ds(0, RPT_LAST)],
                        acc.at[pl.ds(15 * RPT, RPT_LAST)])

    for k in range(AHEAD):
        _iwait(k % NIDX)
        pltpu.async_copy(x_hbm.at[idx_v.at[k % NIDX, 0]], rows[k % NBUF],
                         gsem.at[k % NBUF])

    plsc.subcore_barrier()

    for u in range(UNROLL):
        _step(u, u, swait_ok=(u >= 1), fetch_ok=True, gather_ok=True)

    def body(o, carry):
        for u in range(UNROLL):
            _step(u, o * UNROLL + u, swait_ok=True, fetch_ok=True,
                  gather_ok=True)
        return carry

    lax.fori_loop(1, NMAIN // UNROLL, body, 0)
    for t in range(NPEEL):
        j = NMAIN + t
        _step(t, j, swait_ok=True, fetch_ok=(j + FETCH < NCH),
              gather_ok=(j + AHEAD < NCH))
    for b in range(NBUF):
        _swait(b)
    plsc.subcore_barrier()

    @pl.when(s < 15)
    def _():
        pltpu.sync_copy(acc.at[pl.ds(off, RPT)],
                        out_hbm.at[c, pl.ds(off, RPT)])

    @pl.when(s == 15)
    def _():
        pltpu.sync_copy(acc.at[pl.ds(15 * RPT, RPT_LAST)],
                        out_hbm.at[c, pl.ds(15 * RPT, RPT_LAST)])


def _mlp_body(p_ref, wa_ref, ba_ref, wb_ref, bb_ref, o_ref):
    z = p_ref[0] + p_ref[1]
    h = jnp.dot(z, wa_ref[...], preferred_element_type=jnp.float32,
                precision=lax.Precision.DEFAULT) + ba_ref[...]
    h = jnp.maximum(h, 0.0)
    h = jnp.dot(h, wb_ref[...], preferred_element_type=jnp.float32,
                precision=lax.Precision.DEFAULT) + bb_ref[...]
    o_ref[...] = jnp.maximum(h, 0.0)


_mlp_call = pl.pallas_call(
    _mlp_body,
    out_shape=jax.ShapeDtypeStruct((N, H), jnp.float32),
)


def _head_body(p_ref, wa_ref, ba_ref, wb_ref, bb_ref, batch_ref, wf_ref,
               bf_ref, o_ref):
    z = p_ref[0] + p_ref[1]
    h = jnp.dot(z, wa_ref[...], preferred_element_type=jnp.float32,
                precision=lax.Precision.DEFAULT) + ba_ref[...]
    h = jnp.maximum(h, 0.0)
    h = jnp.dot(h, wb_ref[...], preferred_element_type=jnp.float32,
                precision=lax.Precision.DEFAULT) + bb_ref[...]
    h = jnp.maximum(h, 0.0)
    gids = lax.broadcasted_iota(jnp.int32, (G, 1), 0)
    mask = (gids == batch_ref[...]).astype(jnp.float32)
    pooled = jnp.dot(mask, h, preferred_element_type=jnp.float32,
                     precision=lax.Precision.DEFAULT)
    logits = jnp.dot(pooled, wf_ref[...], preferred_element_type=jnp.float32,
                     precision=lax.Precision.DEFAULT) + bf_ref[...]
    m = jnp.max(logits, axis=-1, keepdims=True)
    e = jnp.exp(logits - m)
    lse = jnp.log(jnp.sum(e, axis=-1, keepdims=True)) + m
    o_ref[...] = logits - lse


_head_call = pl.pallas_call(
    _head_body,
    out_shape=jax.ShapeDtypeStruct((G, OUT), jnp.float32),
)


def kernel(x, edge_index, batch, W1a, b1a, W1b, b1b, W2a, b2a, W2b, b2b,
           Wf, bf):
    src = edge_index[0].reshape(NTILES, NCH, 1, CHUNK)
    dst = edge_index[1].reshape(NTILES, NCH, 1, CHUNK)
    idx = jnp.concatenate([src, dst], axis=2)
    zeros = jnp.zeros((RPT, D), jnp.float32)

    p1 = _scatter_agg(x, idx, zeros)
    h1 = _mlp_call(p1, W1a, b1a.reshape(1, H), W1b, b1b.reshape(1, H))
    p2 = _scatter_agg(h1, idx, zeros)
    return _head_call(p2, W2a, b2a.reshape(1, H), W2b, b2b.reshape(1, H),
                      batch.reshape(1, N), Wf, bf.reshape(1, OUT))
